# baseline (speedup 1.0000x reference)
import functools

import jax
import jax.numpy as jnp
from jax import lax
from jax.experimental import pallas as pl
from jax.experimental.pallas import tpu as pltpu
from jax.experimental.pallas import tpu_sc as plsc

NUM_CLASSES = 21
VAR0 = 0.1
VAR1 = 0.2
THRESHOLD = 0.5
NEGPOS_RATIO = 3

LN2 = 0.6931471805599453
SQRT2 = 1.4142135623730951

TBLK = 10
CATBLK = 112
MINEBLK = 6


def _ln16(x):
    b = lax.bitcast_convert_type(x, jnp.int32)
    e = lax.shift_right_logical(b, 23) - 127
    mb = (b & 0x7FFFFF) | 0x3F800000
    m = lax.bitcast_convert_type(mb, jnp.float32)
    big = m > SQRT2
    m = jnp.where(big, m * 0.5, m)
    e = e + jnp.where(big, 1, 0)
    t = (m - 1.0) / (m + 1.0)
    t2 = t * t
    p = 2.0 * t * (1.0 + t2 * (1.0 / 3.0 + t2 * (0.2 + t2 * (1.0 / 7.0
                                                            + t2 / 9.0))))
    return p + e.astype(jnp.float32) * LN2


def _loss_kernel(db_hbm, loc_hbm, tgt_hbm, cat_hbm, ctail_hbm, misc_hbm,
                 t_v, db0, db1, db2, db3, lb0_v, lb1_v,
                 mrg_v, cat0_v, cat1_v, ctail_v, misc_v,
                 sem0, sem1, sem2, sem3, sem4, *, P, PPAD, O, C):
    NCHUNK = PPAD // 16
    b = lax.axis_index("s") * 2 + lax.axis_index("c")

    pltpu.sync_copy(tgt_hbm.at[b], t_v)
    pltpu.sync_copy(db_hbm.at[0], db0)
    pltpu.sync_copy(db_hbm.at[1], db1)
    pltpu.sync_copy(db_hbm.at[2], db2)
    pltpu.sync_copy(db_hbm.at[3], db3)

    iota16 = lax.iota(jnp.int32, 16)
    negone = jnp.full((16,), -1.0, jnp.float32)
    zero16f = jnp.zeros((16,), jnp.float32)
    zero16i = jnp.zeros((16,), jnp.int32)

    def _init(c, _):
        mrg_v[pl.ds(c * 16, 16)] = negone
        mrg_v[pl.ds(PPAD + c * 16, 16)] = zero16f
        return 0

    lax.fori_loop(0, NCHUNK, _init, 0)

    def _tblock(bk, _):
        t0 = bk * TBLK
        xs1, ys1, xs2, ys2, aas = [], [], [], [], []
        for j in range(TBLK):
            row = t_v[pl.ds((t0 + j) * 8, 16)]
            xs1.append(row[0])
            ys1.append(row[1])
            xs2.append(row[2])
            ys2.append(row[3])
            aas.append((row[2] - row[0]) * (row[3] - row[1]))

        def _chunk(c, carry):
            bpvs = list(carry[:TBLK])
            bpcs = list(carry[TBLK:])
            base = c * 16
            pcx = db0[pl.ds(base, 16)]
            pcy = db1[pl.ds(base, 16)]
            pw = db2[pl.ds(base, 16)]
            ph = db3[pl.ds(base, 16)]
            px1 = pcx - pw * 0.5
            px2 = pcx + pw * 0.5
            py1 = pcy - ph * 0.5
            py2 = pcy + ph * 0.5
            ab = (px2 - px1) * (py2 - py1)
            bto = mrg_v[pl.ds(base, 16)]
            bti = lax.bitcast_convert_type(
                mrg_v[pl.ds(PPAD + base, 16)], jnp.int32)
            ious = []
            for j in range(TBLK):
                iw = jnp.maximum(
                    jnp.minimum(xs2[j], px2) - jnp.maximum(xs1[j], px1), 0.0)
                ih = jnp.maximum(
                    jnp.minimum(ys2[j], py2) - jnp.maximum(ys1[j], py1), 0.0)
                inter = iw * ih
                iou = inter / ((aas[j] + ab) - inter)
                ious.append(iou)
                m2 = iou > bpvs[j]
                bpvs[j] = jnp.where(m2, iou, bpvs[j])
                bpcs[j] = jnp.where(m2, c, bpcs[j])
            pairs = [(ious[j], jnp.full((16,), t0 + j, jnp.int32))
                     for j in range(TBLK)]
            while len(pairs) > 1:
                nxt = []
                for i2 in range(0, len(pairs) - 1, 2):
                    (v1, i1), (v2, j2) = pairs[i2], pairs[i2 + 1]
                    mm = v2 > v1
                    nxt.append((jnp.where(mm, v2, v1),
                                jnp.where(mm, j2, i1)))
                if len(pairs) % 2:
                    nxt.append(pairs[-1])
                pairs = nxt
            bv, bidx = pairs[0]
            m = bv > bto
            bti = jnp.where(m, bidx, bti)
            bto = jnp.where(m, bv, bto)
            mrg_v[pl.ds(base, 16)] = bto
            mrg_v[pl.ds(PPAD + base, 16)] = lax.bitcast_convert_type(
                bti, jnp.float32)
            return tuple(bpvs) + tuple(bpcs)

        carry0 = (negone,) * TBLK + (zero16i,) * TBLK
        carry = lax.fori_loop(0, NCHUNK, _chunk, carry0, unroll=2)

        for j in range(TBLK):
            bpv = carry[j]
            bpc = carry[TBLK + j]
            mx = jnp.max(bpv)
            gidx = bpc * 16 + iota16
            cand = jnp.where(bpv == mx, gidx, jnp.int32(0x7FFFFFF))
            pstar = jnp.full((16,), jnp.min(cand), jnp.int32)
            plsc.store_scatter(mrg_v, [pstar],
                               jnp.full((16,), 2.0, jnp.float32))
            plsc.store_scatter(
                mrg_v, [pstar + PPAD],
                lax.bitcast_convert_type(
                    jnp.full((16,), t0 + j, jnp.int32), jnp.float32))
        return 0

    lax.fori_loop(0, O // TBLK, _tblock, 0)

    NCAT = PPAD // CATBLK
    SUBC = CATBLK // 16

    def _cat_dma(blk, buf, sem):
        return pltpu.async_copy(cat_hbm.at[b, pl.ds(blk * CATBLK, CATBLK)],
                                buf, sem)

    def _loc_dma(blk, buf, sem):
        return pltpu.async_copy(loc_hbm.at[b, blk], buf, sem)

    def _process(blk, buf, lbuf, carry):
        def _sub(sub, carry2):
            sl_acc, np_acc, cep_acc = carry2
            base = blk * CATBLK + sub * 16
            bti_c = lax.bitcast_convert_type(
                mrg_v[pl.ds(PPAD + base, 16)], jnp.int32)
            pos = jnp.logical_not(mrg_v[pl.ds(base, 16)] < THRESHOLD)
            trow = bti_c * 8
            mx1 = plsc.load_gather(t_v, [trow])
            my1 = plsc.load_gather(t_v, [trow + 1])
            mx2 = plsc.load_gather(t_v, [trow + 2])
            my2 = plsc.load_gather(t_v, [trow + 3])
            mlab = plsc.load_gather(t_v, [trow + 4])
            pcx = db0[pl.ds(base, 16)]
            pcy = db1[pl.ds(base, 16)]
            pw = db2[pl.ds(base, 16)]
            ph = db3[pl.ds(base, 16)]
            g_cx = ((mx1 + mx2) * 0.5 - pcx) / (VAR0 * pw)
            g_cy = ((my1 + my2) * 0.5 - pcy) / (VAR0 * ph)
            g_w = _ln16((mx2 - mx1) / pw) / VAR1
            g_h = _ln16((my2 - my1) / ph) / VAR1

            def _sl1(d):
                a = jnp.abs(d)
                return jnp.where(a < 1.0, 0.5 * d * d, a - 0.5)

            lbase = (iota16 + sub * 16) * 4
            s4 = (_sl1(plsc.load_gather(lbuf, [lbase]) - g_cx)
                  + _sl1(plsc.load_gather(lbuf, [lbase + 1]) - g_cy)
                  + _sl1(plsc.load_gather(lbuf, [lbase + 2]) - g_w)
                  + _sl1(plsc.load_gather(lbuf, [lbase + 3]) - g_h))
            sl_acc = sl_acc + jnp.where(pos, s4, 0.0)
            np_acc = np_acc + jnp.where(pos, 1, 0)

            rows = iota16 + sub * 16
            vs = [plsc.load_gather(buf, [rows, jnp.full((16,), c2,
                                                        jnp.int32)])
                  for c2 in range(C)]
            mx = vs
            while len(mx) > 1:
                mx = ([jnp.maximum(mx[i2], mx[i2 + 1])
                       for i2 in range(0, len(mx) - 1, 2)]
                      + ([mx[-1]] if len(mx) % 2 else []))
            mval = mx[0]
            es = [jnp.exp(v - mval) for v in vs]
            while len(es) > 1:
                es = ([es[i2] + es[i2 + 1]
                       for i2 in range(0, len(es) - 1, 2)]
                      + ([es[-1]] if len(es) % 2 else []))
            ssum = es[0]
            logz = _ln16(ssum) + mval
            conf_i = jnp.where(pos, (mlab + 1.0).astype(jnp.int32), 0)
            gt = plsc.load_gather(buf, [rows, conf_i])
            ce_all = logz - gt
            valid = (base + iota16) < P
            ce_mined = jnp.where(
                pos | jnp.logical_not(valid), 0.0, jnp.maximum(ce_all, 0.0))
            mrg_v[pl.ds(2 * PPAD + base, 16)] = ce_mined
            cep_acc = cep_acc + jnp.where(pos, ce_all, 0.0)
            return sl_acc, np_acc, cep_acc

        return lax.fori_loop(0, SUBC, _sub, carry)

    NMAIN = P // CATBLK
    _cat_dma(0, cat0_v, sem0)
    _loc_dma(0, lb0_v, sem3)
    pltpu.async_copy(ctail_hbm.at[b], ctail_v, sem2)

    def _p2(k, carry):
        blk0 = k * 2
        d1 = _cat_dma(blk0 + 1, cat1_v, sem1)
        d1l = _loc_dma(blk0 + 1, lb1_v, sem4)
        pltpu.make_async_copy(cat_hbm.at[b, pl.ds(blk0 * CATBLK, CATBLK)],
                              cat0_v, sem0).wait()
        pltpu.make_async_copy(loc_hbm.at[b, blk0], lb0_v, sem3).wait()
        carry = _process(blk0, cat0_v, lb0_v, carry)

        @pl.when(blk0 + 2 <= NMAIN - 1)
        def _():
            _cat_dma(blk0 + 2, cat0_v, sem0)
            _loc_dma(blk0 + 2, lb0_v, sem3)

        d1.wait()
        d1l.wait()
        carry = _process(blk0 + 1, cat1_v, lb1_v, carry)
        return carry

    carry = lax.fori_loop(0, (NMAIN - 1) // 2, _p2,
                          (zero16f, zero16i, zero16f))
    pltpu.make_async_copy(
        cat_hbm.at[b, pl.ds((NMAIN - 1) * CATBLK, CATBLK)],
        cat0_v, sem0).wait()
    pltpu.make_async_copy(loc_hbm.at[b, NMAIN - 1], lb0_v, sem3).wait()
    carry = _process(NMAIN - 1, cat0_v, lb0_v, carry)
    _loc_dma(NMAIN, lb1_v, sem4).wait()
    pltpu.make_async_copy(ctail_hbm.at[b], ctail_v, sem2).wait()
    sl_acc, np_acc, cep_acc = _process(NMAIN, ctail_v, lb1_v, carry)

    loss_l = jnp.sum(sl_acc)
    npos = jnp.sum(np_acc)
    cepos = jnp.sum(cep_acc)
    k_neg = jnp.minimum(NEGPOS_RATIO * npos, P - 1)

    NSTRIP = PPAD // (16 * MINEBLK)

    def _bit_step(j, cand):
        test = jnp.full((16,), cand | (1 << (30 - j)), jnp.int32)

        def _cnt(i2, accs):
            a0, a1, a2 = accs
            parts = []
            for u in range(MINEBLK):
                v = lax.bitcast_convert_type(
                    mrg_v[pl.ds(2 * PPAD + (i2 * MINEBLK + u) * 16, 16)],
                    jnp.int32)
                parts.append(plsc.all_reduce_population_count(v >= test))
            for u in range(0, MINEBLK, 3):
                a0 = a0 + parts[u]
                a1 = a1 + parts[u + 1]
                a2 = a2 + parts[u + 2]
            return a0, a1, a2

        c0, c1, c2m = lax.fori_loop(0, NSTRIP, _cnt,
                                    (zero16i, zero16i, zero16i))
        cnt = (c0 + c1 + c2m)[0]
        return jnp.where(cnt >= k_neg, cand | (1 << (30 - j)), cand)

    tbits = lax.fori_loop(0, 31, _bit_step, jnp.int32(0))
    tval = lax.bitcast_convert_type(tbits, jnp.float32)
    tvec = jnp.full((16,), tbits, jnp.int32)

    def _final(i2, carry):
        cnt_acc, sum_acc = carry
        for u in range(MINEBLK):
            vb = lax.bitcast_convert_type(
                mrg_v[pl.ds(2 * PPAD + (i2 * MINEBLK + u) * 16, 16)],
                jnp.int32)
            gtm = vb > tvec
            cnt_acc = cnt_acc + plsc.all_reduce_population_count(gtm)
            sum_acc = sum_acc + jnp.where(
                gtm, lax.bitcast_convert_type(vb, jnp.float32), 0.0)
        return cnt_acc, sum_acc

    cnt_gt16, sum_gt16 = lax.fori_loop(0, NSTRIP, _final, (zero16i, zero16f))
    cnt_gt = cnt_gt16[0]
    sum_gt = jnp.sum(sum_gt16)
    topk = sum_gt + (k_neg - cnt_gt).astype(jnp.float32) * tval
    loss_c = cepos + topk

    misc_v[...] = (jnp.where(iota16 == 0, loss_l, 0.0)
                   + jnp.where(iota16 == 1, loss_c, 0.0)
                   + jnp.where(iota16 == 2, npos.astype(jnp.float32), 0.0))
    pltpu.sync_copy(misc_v, misc_hbm.at[b])


def _combine_kernel(misc_ref, out_l_ref, out_c_ref):
    m = misc_ref[...]
    n = jnp.sum(m[:, 2])
    out_l_ref[0, 0] = jnp.sum(m[:, 0]) / n
    out_c_ref[0, 0] = jnp.sum(m[:, 1]) / n


def kernel(location, category, defaultbox, targets):
    B, P, C = category.shape
    O = targets.shape[1]
    PPAD = ((P + 15) // 16) * 16

    db_t = jnp.transpose(defaultbox, (1, 0))
    pad_col = jnp.array([2.0, 2.0, 0.01, 0.01], jnp.float32)[:, None]
    db_pad = jnp.concatenate(
        [db_t, jnp.broadcast_to(pad_col, (4, PPAD - P))], axis=1)
    loc_pad = jnp.concatenate(
        [location, jnp.zeros((B, PPAD - P, 4), jnp.float32)],
        axis=1).reshape(B, PPAD // CATBLK, CATBLK * 4)
    tgt_pad = jnp.concatenate(
        [targets, jnp.zeros((B, O, 3), jnp.float32)], axis=2)
    tgt_pad = jnp.concatenate(
        [tgt_pad, jnp.zeros((B, 2, 8), jnp.float32)],
        axis=1).reshape(B, (O + 2) * 8)

    mesh = plsc.VectorSubcoreMesh(core_axis_name="c", subcore_axis_name="s")
    loss = functools.partial(
        pl.kernel,
        mesh=mesh,
        compiler_params=pltpu.CompilerParams(needs_layout_passes=False),
        out_type=[
            jax.ShapeDtypeStruct((B, 16), jnp.float32),
        ],
        scratch_types=[
            pltpu.VMEM(((O + 2) * 8,), jnp.float32),
            pltpu.VMEM((PPAD,), jnp.float32),
            pltpu.VMEM((PPAD,), jnp.float32),
            pltpu.VMEM((PPAD,), jnp.float32),
            pltpu.VMEM((PPAD,), jnp.float32),
            pltpu.VMEM((CATBLK * 4,), jnp.float32),
            pltpu.VMEM((CATBLK * 4,), jnp.float32),
            pltpu.VMEM((3 * PPAD,), jnp.float32),
            pltpu.VMEM((CATBLK, NUM_CLASSES), jnp.float32),
            pltpu.VMEM((CATBLK, NUM_CLASSES), jnp.float32),
            pltpu.VMEM((CATBLK, NUM_CLASSES), jnp.float32),
            pltpu.VMEM((16,), jnp.float32),
            pltpu.SemaphoreType.DMA,
            pltpu.SemaphoreType.DMA,
            pltpu.SemaphoreType.DMA,
            pltpu.SemaphoreType.DMA,
            pltpu.SemaphoreType.DMA,
        ],
    )(functools.partial(_loss_kernel, P=P, PPAD=PPAD, O=O, C=C))
    nmain = P // CATBLK
    cat_tail = jnp.concatenate(
        [category[:, nmain * CATBLK:, :],
         jnp.zeros((B, CATBLK - (P - nmain * CATBLK), C), jnp.float32)],
        axis=1)
    (misc,) = loss(db_pad, loc_pad, tgt_pad, category, cat_tail)

    out_l, out_c = pl.pallas_call(
        _combine_kernel,
        out_specs=[
            pl.BlockSpec(memory_space=pltpu.SMEM),
            pl.BlockSpec(memory_space=pltpu.SMEM),
        ],
        out_shape=[
            jax.ShapeDtypeStruct((1, 1), jnp.float32),
            jax.ShapeDtypeStruct((1, 1), jnp.float32),
        ],
    )(misc)
    return out_l[0, 0], out_c[0, 0]


# speedup vs baseline: 1.1517x; 1.1517x over previous
import functools

import jax
import jax.numpy as jnp
from jax import lax
from jax.experimental import pallas as pl
from jax.experimental.pallas import tpu as pltpu
from jax.experimental.pallas import tpu_sc as plsc

NUM_CLASSES = 21
VAR0 = 0.1
VAR1 = 0.2
THRESHOLD = 0.5
NEGPOS_RATIO = 3

LN2 = 0.6931471805599453
SQRT2 = 1.4142135623730951

TBLK = 10
CATBLK = 112
MINEBLK = 21


def _ln16(x):
    b = lax.bitcast_convert_type(x, jnp.int32)
    e = lax.shift_right_logical(b, 23) - 127
    mb = (b & 0x7FFFFF) | 0x3F800000
    m = lax.bitcast_convert_type(mb, jnp.float32)
    big = m > SQRT2
    m = jnp.where(big, m * 0.5, m)
    e = e + jnp.where(big, 1, 0)
    t = (m - 1.0) / (m + 1.0)
    t2 = t * t
    p = 2.0 * t * (1.0 + t2 * (1.0 / 3.0 + t2 * (0.2 + t2 * (1.0 / 7.0
                                                            + t2 / 9.0))))
    return p + e.astype(jnp.float32) * LN2


def _loss_kernel(db_hbm, loc_hbm, tgt_hbm, cat_hbm, ctail_hbm, misc_hbm,
                 t_v, db0, db1, db2, db3, lb0_v, lb1_v,
                 mrg_v, cat0_v, cat1_v, ctail_v, misc_v,
                 sem0, sem1, sem2, sem3, sem4, *, P, PPAD, O, C):
    NCHUNK = PPAD // 16
    b = lax.axis_index("s") * 2 + lax.axis_index("c")

    pltpu.sync_copy(tgt_hbm.at[b], t_v)
    pltpu.sync_copy(db_hbm.at[0], db0)
    pltpu.sync_copy(db_hbm.at[1], db1)
    pltpu.sync_copy(db_hbm.at[2], db2)
    pltpu.sync_copy(db_hbm.at[3], db3)

    iota16 = lax.iota(jnp.int32, 16)
    negone = jnp.full((16,), -1.0, jnp.float32)
    zero16f = jnp.zeros((16,), jnp.float32)
    zero16i = jnp.zeros((16,), jnp.int32)

    def _init(c, _):
        mrg_v[pl.ds(c * 16, 16)] = negone
        mrg_v[pl.ds(PPAD + c * 16, 16)] = zero16f
        return 0

    lax.fori_loop(0, NCHUNK, _init, 0)

    def _tblock(bk, _):
        t0 = bk * TBLK
        xs1, ys1, xs2, ys2, aas = [], [], [], [], []
        for j in range(TBLK):
            row = t_v[pl.ds((t0 + j) * 8, 16)]
            xs1.append(row[0])
            ys1.append(row[1])
            xs2.append(row[2])
            ys2.append(row[3])
            aas.append((row[2] - row[0]) * (row[3] - row[1]))

        def _chunk(c, carry):
            bpvs = list(carry[:TBLK])
            bpcs = list(carry[TBLK:])
            base = c * 16
            pcx = db0[pl.ds(base, 16)]
            pcy = db1[pl.ds(base, 16)]
            pw = db2[pl.ds(base, 16)]
            ph = db3[pl.ds(base, 16)]
            px1 = pcx - pw * 0.5
            px2 = pcx + pw * 0.5
            py1 = pcy - ph * 0.5
            py2 = pcy + ph * 0.5
            ab = (px2 - px1) * (py2 - py1)
            bto = mrg_v[pl.ds(base, 16)]
            bti = lax.bitcast_convert_type(
                mrg_v[pl.ds(PPAD + base, 16)], jnp.int32)
            ious = []
            for j in range(TBLK):
                iw = jnp.maximum(
                    jnp.minimum(xs2[j], px2) - jnp.maximum(xs1[j], px1), 0.0)
                ih = jnp.maximum(
                    jnp.minimum(ys2[j], py2) - jnp.maximum(ys1[j], py1), 0.0)
                inter = iw * ih
                iou = inter / ((aas[j] + ab) - inter)
                ious.append(iou)
                m2 = iou > bpvs[j]
                bpvs[j] = jnp.where(m2, iou, bpvs[j])
                bpcs[j] = jnp.where(m2, c, bpcs[j])
            pairs = [(ious[j], jnp.full((16,), t0 + j, jnp.int32))
                     for j in range(TBLK)]
            while len(pairs) > 1:
                nxt = []
                for i2 in range(0, len(pairs) - 1, 2):
                    (v1, i1), (v2, j2) = pairs[i2], pairs[i2 + 1]
                    mm = v2 > v1
                    nxt.append((jnp.where(mm, v2, v1),
                                jnp.where(mm, j2, i1)))
                if len(pairs) % 2:
                    nxt.append(pairs[-1])
                pairs = nxt
            bv, bidx = pairs[0]
            m = bv > bto
            bti = jnp.where(m, bidx, bti)
            bto = jnp.where(m, bv, bto)
            mrg_v[pl.ds(base, 16)] = bto
            mrg_v[pl.ds(PPAD + base, 16)] = lax.bitcast_convert_type(
                bti, jnp.float32)
            return tuple(bpvs) + tuple(bpcs)

        carry0 = (negone,) * TBLK + (zero16i,) * TBLK
        carry = lax.fori_loop(0, NCHUNK, _chunk, carry0)

        for j in range(TBLK):
            bpv = carry[j]
            bpc = carry[TBLK + j]
            mx = jnp.max(bpv)
            gidx = bpc * 16 + iota16
            cand = jnp.where(bpv == mx, gidx, jnp.int32(0x7FFFFFF))
            pstar = jnp.full((16,), jnp.min(cand), jnp.int32)
            plsc.store_scatter(mrg_v, [pstar],
                               jnp.full((16,), 2.0, jnp.float32))
            plsc.store_scatter(
                mrg_v, [pstar + PPAD],
                lax.bitcast_convert_type(
                    jnp.full((16,), t0 + j, jnp.int32), jnp.float32))
        return 0

    lax.fori_loop(0, O // TBLK, _tblock, 0)

    NCAT = PPAD // CATBLK
    SUBC = CATBLK // 16

    def _cat_dma(blk, buf, sem):
        return pltpu.async_copy(cat_hbm.at[b, pl.ds(blk * CATBLK, CATBLK)],
                                buf, sem)

    def _loc_dma(blk, buf, sem):
        return pltpu.async_copy(loc_hbm.at[b, blk], buf, sem)

    def _process(blk, buf, lbuf, carry):
        def _sub(sub, carry2):
            sl_acc, np_acc, cep_acc = carry2
            base = blk * CATBLK + sub * 16
            bti_c = lax.bitcast_convert_type(
                mrg_v[pl.ds(PPAD + base, 16)], jnp.int32)
            pos = jnp.logical_not(mrg_v[pl.ds(base, 16)] < THRESHOLD)
            trow = bti_c * 8
            mx1 = plsc.load_gather(t_v, [trow])
            my1 = plsc.load_gather(t_v, [trow + 1])
            mx2 = plsc.load_gather(t_v, [trow + 2])
            my2 = plsc.load_gather(t_v, [trow + 3])
            mlab = plsc.load_gather(t_v, [trow + 4])
            pcx = db0[pl.ds(base, 16)]
            pcy = db1[pl.ds(base, 16)]
            pw = db2[pl.ds(base, 16)]
            ph = db3[pl.ds(base, 16)]
            g_cx = ((mx1 + mx2) * 0.5 - pcx) / (VAR0 * pw)
            g_cy = ((my1 + my2) * 0.5 - pcy) / (VAR0 * ph)
            g_w = _ln16((mx2 - mx1) / pw) / VAR1
            g_h = _ln16((my2 - my1) / ph) / VAR1

            def _sl1(d):
                a = jnp.abs(d)
                return jnp.where(a < 1.0, 0.5 * d * d, a - 0.5)

            lbase = (iota16 + sub * 16) * 4
            s4 = (_sl1(plsc.load_gather(lbuf, [lbase]) - g_cx)
                  + _sl1(plsc.load_gather(lbuf, [lbase + 1]) - g_cy)
                  + _sl1(plsc.load_gather(lbuf, [lbase + 2]) - g_w)
                  + _sl1(plsc.load_gather(lbuf, [lbase + 3]) - g_h))
            sl_acc = sl_acc + jnp.where(pos, s4, 0.0)
            np_acc = np_acc + jnp.where(pos, 1, 0)

            rows = iota16 + sub * 16
            vs = [plsc.load_gather(buf, [rows, jnp.full((16,), c2,
                                                        jnp.int32)])
                  for c2 in range(C)]
            mx = vs
            while len(mx) > 1:
                mx = ([jnp.maximum(mx[i2], mx[i2 + 1])
                       for i2 in range(0, len(mx) - 1, 2)]
                      + ([mx[-1]] if len(mx) % 2 else []))
            mval = mx[0]
            es = [jnp.exp(v - mval) for v in vs]
            while len(es) > 1:
                es = ([es[i2] + es[i2 + 1]
                       for i2 in range(0, len(es) - 1, 2)]
                      + ([es[-1]] if len(es) % 2 else []))
            ssum = es[0]
            logz = _ln16(ssum) + mval
            conf_i = jnp.where(pos, (mlab + 1.0).astype(jnp.int32), 0)
            gt = plsc.load_gather(buf, [rows, conf_i])
            ce_all = logz - gt
            valid = (base + iota16) < P
            ce_mined = jnp.where(
                pos | jnp.logical_not(valid), 0.0, jnp.maximum(ce_all, 0.0))
            mrg_v[pl.ds(2 * PPAD + base, 16)] = ce_mined
            cep_acc = cep_acc + jnp.where(pos, ce_all, 0.0)
            return sl_acc, np_acc, cep_acc

        return lax.fori_loop(0, SUBC, _sub, carry)

    NMAIN = P // CATBLK
    _cat_dma(0, cat0_v, sem0)
    _loc_dma(0, lb0_v, sem3)
    pltpu.async_copy(ctail_hbm.at[b], ctail_v, sem2)

    def _p2(k, carry):
        blk0 = k * 2
        d1 = _cat_dma(blk0 + 1, cat1_v, sem1)
        d1l = _loc_dma(blk0 + 1, lb1_v, sem4)
        pltpu.make_async_copy(cat_hbm.at[b, pl.ds(blk0 * CATBLK, CATBLK)],
                              cat0_v, sem0).wait()
        pltpu.make_async_copy(loc_hbm.at[b, blk0], lb0_v, sem3).wait()
        carry = _process(blk0, cat0_v, lb0_v, carry)

        @pl.when(blk0 + 2 <= NMAIN - 1)
        def _():
            _cat_dma(blk0 + 2, cat0_v, sem0)
            _loc_dma(blk0 + 2, lb0_v, sem3)

        d1.wait()
        d1l.wait()
        carry = _process(blk0 + 1, cat1_v, lb1_v, carry)
        return carry

    carry = lax.fori_loop(0, (NMAIN - 1) // 2, _p2,
                          (zero16f, zero16i, zero16f))
    pltpu.make_async_copy(
        cat_hbm.at[b, pl.ds((NMAIN - 1) * CATBLK, CATBLK)],
        cat0_v, sem0).wait()
    pltpu.make_async_copy(loc_hbm.at[b, NMAIN - 1], lb0_v, sem3).wait()
    carry = _process(NMAIN - 1, cat0_v, lb0_v, carry)
    _loc_dma(NMAIN, lb1_v, sem4).wait()
    pltpu.make_async_copy(ctail_hbm.at[b], ctail_v, sem2).wait()
    sl_acc, np_acc, cep_acc = _process(NMAIN, ctail_v, lb1_v, carry)

    loss_l = jnp.sum(sl_acc)
    npos = jnp.sum(np_acc)
    cepos = jnp.sum(cep_acc)
    k_neg = jnp.minimum(NEGPOS_RATIO * npos, P - 1)

    NSTRIP = PPAD // (16 * MINEBLK)

    def _bit_step(j, cand):
        test = jnp.full((16,), cand | (1 << (30 - j)), jnp.int32)

        def _cnt(i2, accs):
            a0, a1, a2 = accs
            parts = []
            for u in range(MINEBLK):
                v = lax.bitcast_convert_type(
                    mrg_v[pl.ds(2 * PPAD + (i2 * MINEBLK + u) * 16, 16)],
                    jnp.int32)
                parts.append(plsc.all_reduce_population_count(v >= test))
            for u in range(0, MINEBLK, 3):
                a0 = a0 + parts[u]
                a1 = a1 + parts[u + 1]
                a2 = a2 + parts[u + 2]
            return a0, a1, a2

        c0, c1, c2m = lax.fori_loop(0, NSTRIP, _cnt,
                                    (zero16i, zero16i, zero16i))
        cnt = (c0 + c1 + c2m)[0]
        return jnp.where(cnt >= k_neg, cand | (1 << (30 - j)), cand)

    tbits = lax.fori_loop(0, 31, _bit_step, jnp.int32(0))
    tval = lax.bitcast_convert_type(tbits, jnp.float32)
    tvec = jnp.full((16,), tbits, jnp.int32)

    def _final(i2, carry):
        cnt_acc, sum_acc = carry
        for u in range(MINEBLK):
            vb = lax.bitcast_convert_type(
                mrg_v[pl.ds(2 * PPAD + (i2 * MINEBLK + u) * 16, 16)],
                jnp.int32)
            gtm = vb > tvec
            cnt_acc = cnt_acc + plsc.all_reduce_population_count(gtm)
            sum_acc = sum_acc + jnp.where(
                gtm, lax.bitcast_convert_type(vb, jnp.float32), 0.0)
        return cnt_acc, sum_acc

    cnt_gt16, sum_gt16 = lax.fori_loop(0, NSTRIP, _final, (zero16i, zero16f))
    cnt_gt = cnt_gt16[0]
    sum_gt = jnp.sum(sum_gt16)
    topk = sum_gt + (k_neg - cnt_gt).astype(jnp.float32) * tval
    loss_c = cepos + topk

    misc_v[...] = (jnp.where(iota16 == 0, loss_l, 0.0)
                   + jnp.where(iota16 == 1, loss_c, 0.0)
                   + jnp.where(iota16 == 2, npos.astype(jnp.float32), 0.0))
    pltpu.sync_copy(misc_v, misc_hbm.at[b])


def _combine_kernel(misc_ref, out_l_ref, out_c_ref):
    m = misc_ref[...]
    n = jnp.sum(m[:, 2])
    out_l_ref[0, 0] = jnp.sum(m[:, 0]) / n
    out_c_ref[0, 0] = jnp.sum(m[:, 1]) / n


def kernel(location, category, defaultbox, targets):
    B, P, C = category.shape
    O = targets.shape[1]
    PPAD = ((P + 15) // 16) * 16

    db_t = jnp.transpose(defaultbox, (1, 0))
    pad_col = jnp.array([2.0, 2.0, 0.01, 0.01], jnp.float32)[:, None]
    db_pad = jnp.concatenate(
        [db_t, jnp.broadcast_to(pad_col, (4, PPAD - P))], axis=1)
    loc_pad = jnp.concatenate(
        [location, jnp.zeros((B, PPAD - P, 4), jnp.float32)],
        axis=1).reshape(B, PPAD // CATBLK, CATBLK * 4)
    tgt_pad = jnp.concatenate(
        [targets, jnp.zeros((B, O, 3), jnp.float32)], axis=2)
    tgt_pad = jnp.concatenate(
        [tgt_pad, jnp.zeros((B, 2, 8), jnp.float32)],
        axis=1).reshape(B, (O + 2) * 8)

    mesh = plsc.VectorSubcoreMesh(core_axis_name="c", subcore_axis_name="s")
    loss = functools.partial(
        pl.kernel,
        mesh=mesh,
        compiler_params=pltpu.CompilerParams(needs_layout_passes=False),
        out_type=[
            jax.ShapeDtypeStruct((B, 16), jnp.float32),
        ],
        scratch_types=[
            pltpu.VMEM(((O + 2) * 8,), jnp.float32),
            pltpu.VMEM((PPAD,), jnp.float32),
            pltpu.VMEM((PPAD,), jnp.float32),
            pltpu.VMEM((PPAD,), jnp.float32),
            pltpu.VMEM((PPAD,), jnp.float32),
            pltpu.VMEM((CATBLK * 4,), jnp.float32),
            pltpu.VMEM((CATBLK * 4,), jnp.float32),
            pltpu.VMEM((3 * PPAD,), jnp.float32),
            pltpu.VMEM((CATBLK, NUM_CLASSES), jnp.float32),
            pltpu.VMEM((CATBLK, NUM_CLASSES), jnp.float32),
            pltpu.VMEM((CATBLK, NUM_CLASSES), jnp.float32),
            pltpu.VMEM((16,), jnp.float32),
            pltpu.SemaphoreType.DMA,
            pltpu.SemaphoreType.DMA,
            pltpu.SemaphoreType.DMA,
            pltpu.SemaphoreType.DMA,
            pltpu.SemaphoreType.DMA,
        ],
    )(functools.partial(_loss_kernel, P=P, PPAD=PPAD, O=O, C=C))
    nmain = P // CATBLK
    cat_tail = jnp.concatenate(
        [category[:, nmain * CATBLK:, :],
         jnp.zeros((B, CATBLK - (P - nmain * CATBLK), C), jnp.float32)],
        axis=1)
    (misc,) = loss(db_pad, loc_pad, tgt_pad, category, cat_tail)

    out_l, out_c = pl.pallas_call(
        _combine_kernel,
        out_specs=[
            pl.BlockSpec(memory_space=pltpu.SMEM),
            pl.BlockSpec(memory_space=pltpu.SMEM),
        ],
        out_shape=[
            jax.ShapeDtypeStruct((1, 1), jnp.float32),
            jax.ShapeDtypeStruct((1, 1), jnp.float32),
        ],
    )(misc)
    return out_l[0, 0], out_c[0, 0]
